# baseline (speedup 1.0000x reference)
import functools

import jax
import jax.numpy as jnp
from jax import lax
from jax.experimental import pallas as pl
from jax.experimental.pallas import tpu as pltpu
from jax.experimental.pallas import tpu_sc as plsc

_K = 5
_IGNORE = -100
_B = 128
_C = 100000
_LANES = 16
_NC = 2
_NS = 16
_NW = _NC * _NS
_KV = _B // _LANES

_S_SC = 33280
_SLAB = _S_SC // _NW
_RCH = 208
_NCH = _SLAB // _RCH

_RT = 160
_GT = (_C - _S_SC) // _RT
_TOFF = _S_SC // _RT


def _nextup(v):
    bi = lax.bitcast_convert_type(v, jnp.int32)
    bp = jnp.where(bi < 0, bi - 1, bi + 1)
    bp = jnp.where(bi == jnp.int32(-2147483648), jnp.int32(1), bp)
    return lax.bitcast_convert_type(bp, jnp.float32)


def _sc_body(x_hbm, ytrue_hbm, cnt_hbm,
             yt_v, idx_v, g_v, buf0, buf1, cnt_v,
             sem0, sem1, gsem):
    wid = lax.axis_index("s") * _NC + lax.axis_index("c")
    s0 = wid * _SLAB
    bufs = (buf0, buf1)
    sems = (sem0, sem1)

    def chunk_copy(c):
        off = pl.multiple_of(s0 + c * _RCH, 8)
        return pltpu.async_copy(x_hbm.at[pl.ds(off, _RCH)],
                                bufs[c % 2], sems[c % 2])

    cps = [chunk_copy(0), chunk_copy(1)]

    pltpu.sync_copy(ytrue_hbm, yt_v)
    lane_iota = lax.iota(jnp.int32, _LANES)
    for k in range(_KV):
        idx_v[pl.ds(k * _LANES, _LANES)] = jnp.clip(
            yt_v[pl.ds(k * _LANES, _LANES)], 0, _C - 1)
    pltpu.async_copy(x_hbm.at[idx_v], g_v, gsem).wait()

    ts, vps, thr0s = [], [], []
    for k in range(_KV):
        i16 = k * _LANES + lane_iota
        t_k = yt_v[pl.ds(k * _LANES, _LANES)]
        v_k = plsc.load_gather(g_v, [i16, i16])
        ts.append(t_k)
        vps.append(_nextup(v_k))
        thr0s.append(v_k)

    zero_i = jnp.zeros((_LANES,), jnp.int32)
    accs = (zero_i,) * _KV

    def fast_chunk(buf, c0, accs):
        thrs = [jnp.where(ts[k] >= c0 + _RCH, thr0s[k], vps[k])
                for k in range(_KV)]

        def step(rr, a):
            row = buf.at[rr]
            out = []
            for k in range(_KV):
                x = row[pl.ds(k * _LANES, _LANES)]
                out.append(a[k] + (x >= thrs[k]).astype(jnp.int32))
            return tuple(out)
        return lax.fori_loop(0, _RCH, step, accs)

    def slow_chunk(buf, c0, nrows, accs):
        thrs = tuple(jnp.where(ts[k] > c0, thr0s[k], vps[k])
                     for k in range(_KV))

        def step(rr, carry):
            a, th = carry
            j = jnp.full((_LANES,), c0, jnp.int32) + rr
            row = buf.at[rr]
            na, nth = [], []
            for k in range(_KV):
                x = row[pl.ds(k * _LANES, _LANES)]
                tk = jnp.where(j == ts[k], vps[k], th[k])
                na.append(a[k] + (x >= tk).astype(jnp.int32))
                nth.append(tk)
            return tuple(na), tuple(nth)
        accs, _ = lax.fori_loop(0, nrows, step, (accs, thrs))
        return accs

    for c in range(_NCH):
        cps[c].wait()
        c0 = s0 + c * _RCH
        c0s = jnp.full((_LANES,), c0, jnp.int32)
        inb = zero_i
        for k in range(_KV):
            inb = inb + plsc.all_reduce_population_count(
                (ts[k] >= c0s) & (ts[k] < c0s + _RCH))
        has_t = jnp.max(inb) > 0
        buf = bufs[c % 2]
        accs = lax.cond(
            has_t,
            functools.partial(slow_chunk, buf, c0, _RCH),
            functools.partial(fast_chunk, buf, c0),
            accs)
        if c + 2 < _NCH:
            cps.append(chunk_copy(c + 2))

    for k in range(_KV):
        cnt_v[pl.ds(k * _LANES, _LANES)] = accs[k]
    pltpu.sync_copy(cnt_v, cnt_hbm.at[wid])


def _tc_body(v_ref, t_ref, x_ref, o_ref):
    i = pl.program_id(0)

    @pl.when(i == 0)
    def _():
        o_ref[...] = jnp.zeros_like(o_ref)

    x = x_ref[...]
    v = v_ref[...]
    t = t_ref[...]
    j = lax.broadcasted_iota(jnp.int32, (_RT, _B), 0) + (_S_SC + i * _RT)
    m = (x > v) | ((x == v) & (j < t))
    o_ref[...] += m.astype(jnp.int32).sum(axis=0, keepdims=True)


@jax.jit
def kernel(y_pred, y_true):
    yt = y_true.astype(jnp.int32)
    x_t = y_pred.T

    mesh = plsc.VectorSubcoreMesh(core_axis_name="c", subcore_axis_name="s")
    sc = functools.partial(
        pl.kernel,
        mesh=mesh,
        compiler_params=pltpu.CompilerParams(needs_layout_passes=False),
        out_type=jax.ShapeDtypeStruct((_NW, _B), jnp.int32),
        scratch_types=[
            pltpu.VMEM((_B,), jnp.int32),
            pltpu.VMEM((_B,), jnp.int32),
            pltpu.VMEM((_B, _B), jnp.float32),
            pltpu.VMEM((_RCH, _B), jnp.float32),
            pltpu.VMEM((_RCH, _B), jnp.float32),
            pltpu.VMEM((_B,), jnp.int32),
            pltpu.SemaphoreType.DMA,
            pltpu.SemaphoreType.DMA,
            pltpu.SemaphoreType.DMA,
        ],
    )(_sc_body)
    sc_partials = sc(x_t, yt)

    tt = jnp.clip(yt, 0, _C - 1)
    v = jnp.take_along_axis(y_pred, tt[:, None], axis=1)[:, 0]
    tc_counts = pl.pallas_call(
        _tc_body,
        grid=(_GT,),
        in_specs=[
            pl.BlockSpec((1, _B), lambda i: (0, 0)),
            pl.BlockSpec((1, _B), lambda i: (0, 0)),
            pl.BlockSpec((_RT, _B), lambda i: (i + _TOFF, 0)),
        ],
        out_specs=pl.BlockSpec((1, _B), lambda i: (0, 0)),
        out_shape=jax.ShapeDtypeStruct((1, _B), jnp.int32),
    )(v[None, :], tt[None, :], x_t)

    counts = sc_partials.sum(axis=0) + tc_counts[0]
    valid = y_true != _IGNORE
    hits = ((counts < _K) & valid).astype(jnp.float32)
    w = valid.astype(jnp.float32)
    return (hits.sum() / w.sum()) * 100.0


# speedup vs baseline: 3.9354x; 3.9354x over previous
import functools

import jax
import jax.numpy as jnp
from jax import lax
from jax.experimental import pallas as pl
from jax.experimental.pallas import tpu as pltpu
from jax.experimental.pallas import tpu_sc as plsc

_K = 5
_IGNORE = -100
_B = 128
_C = 100000
_LANES = 16
_NC = 2
_NS = 16
_NW = _NC * _NS
_KV = _B // _LANES

_S_SC = 32000
_SLAB = _S_SC // _NW
_RCH = 200
_NCH = _SLAB // _RCH

_RT = 2000
_GT = (_C - _S_SC) // _RT
_TOFF = _S_SC // _RT


def _nextup(v):
    bi = lax.bitcast_convert_type(v, jnp.int32)
    bp = jnp.where(bi < 0, bi - 1, bi + 1)
    bp = jnp.where(bi == jnp.int32(-2147483648), jnp.int32(1), bp)
    return lax.bitcast_convert_type(bp, jnp.float32)


def _sc_body(x_hbm, ytrue_hbm, cnt_hbm,
             yt_v, idx_v, g_v, buf0, buf1, cnt_v,
             sem0, sem1, gsem):
    wid = lax.axis_index("s") * _NC + lax.axis_index("c")
    s0 = wid * _SLAB
    bufs = (buf0, buf1)
    sems = (sem0, sem1)

    def chunk_copy(c):
        off = pl.multiple_of(s0 + c * _RCH, 8)
        return pltpu.async_copy(x_hbm.at[pl.ds(off, _RCH)],
                                bufs[c % 2], sems[c % 2])

    cps = [chunk_copy(0), chunk_copy(1)]

    pltpu.sync_copy(ytrue_hbm, yt_v)
    lane_iota = lax.iota(jnp.int32, _LANES)
    for k in range(_KV):
        idx_v[pl.ds(k * _LANES, _LANES)] = jnp.clip(
            yt_v[pl.ds(k * _LANES, _LANES)], 0, _C - 1)
    pltpu.async_copy(x_hbm.at[idx_v], g_v, gsem).wait()

    ts, vps, thr0s = [], [], []
    for k in range(_KV):
        i16 = k * _LANES + lane_iota
        t_k = yt_v[pl.ds(k * _LANES, _LANES)]
        v_k = plsc.load_gather(g_v, [i16, i16])
        ts.append(t_k)
        vps.append(_nextup(v_k))
        thr0s.append(v_k)

    zero_i = jnp.zeros((_LANES,), jnp.int32)
    accs = (zero_i,) * _KV

    def fast_chunk(buf, c0, accs):
        thrs = [jnp.where(ts[k] >= c0 + _RCH, thr0s[k], vps[k])
                for k in range(_KV)]

        def step(rr, a):
            row = buf.at[rr]
            out = []
            for k in range(_KV):
                x = row[pl.ds(k * _LANES, _LANES)]
                out.append(a[k] + (x >= thrs[k]).astype(jnp.int32))
            return tuple(out)
        return lax.fori_loop(0, _RCH, step, accs)

    def slow_chunk(buf, c0, nrows, accs):
        thrs = tuple(jnp.where(ts[k] > c0, thr0s[k], vps[k])
                     for k in range(_KV))

        def step(rr, carry):
            a, th = carry
            j = jnp.full((_LANES,), c0, jnp.int32) + rr
            row = buf.at[rr]
            na, nth = [], []
            for k in range(_KV):
                x = row[pl.ds(k * _LANES, _LANES)]
                tk = jnp.where(j == ts[k], vps[k], th[k])
                na.append(a[k] + (x >= tk).astype(jnp.int32))
                nth.append(tk)
            return tuple(na), tuple(nth)
        accs, _ = lax.fori_loop(0, nrows, step, (accs, thrs))
        return accs

    for c in range(_NCH):
        cps[c].wait()
        c0 = s0 + c * _RCH
        c0s = jnp.full((_LANES,), c0, jnp.int32)
        inb = zero_i
        for k in range(_KV):
            inb = inb + plsc.all_reduce_population_count(
                (ts[k] >= c0s) & (ts[k] < c0s + _RCH))
        has_t = jnp.max(inb) > 0
        buf = bufs[c % 2]
        accs = lax.cond(
            has_t,
            functools.partial(slow_chunk, buf, c0, _RCH),
            functools.partial(fast_chunk, buf, c0),
            accs)
        if c + 2 < _NCH:
            cps.append(chunk_copy(c + 2))

    for k in range(_KV):
        cnt_v[pl.ds(k * _LANES, _LANES)] = accs[k]
    pltpu.sync_copy(cnt_v, cnt_hbm.at[wid])


def _tc_body(v_ref, t_ref, x_ref, o_ref):
    i = pl.program_id(0)

    @pl.when(i == 0)
    def _():
        o_ref[...] = jnp.zeros_like(o_ref)

    x = x_ref[...]
    v = v_ref[...]
    t = t_ref[...]
    j = lax.broadcasted_iota(jnp.int32, (_RT, _B), 0) + (_S_SC + i * _RT)
    m = (x > v) | ((x == v) & (j < t))
    o_ref[...] += m.astype(jnp.int32).sum(axis=0, keepdims=True)


@jax.jit
def kernel(y_pred, y_true):
    yt = y_true.astype(jnp.int32)
    x_t = y_pred.T

    mesh = plsc.VectorSubcoreMesh(core_axis_name="c", subcore_axis_name="s")
    sc = functools.partial(
        pl.kernel,
        mesh=mesh,
        compiler_params=pltpu.CompilerParams(needs_layout_passes=False),
        out_type=jax.ShapeDtypeStruct((_NW, _B), jnp.int32),
        scratch_types=[
            pltpu.VMEM((_B,), jnp.int32),
            pltpu.VMEM((_B,), jnp.int32),
            pltpu.VMEM((_B, _B), jnp.float32),
            pltpu.VMEM((_RCH, _B), jnp.float32),
            pltpu.VMEM((_RCH, _B), jnp.float32),
            pltpu.VMEM((_B,), jnp.int32),
            pltpu.SemaphoreType.DMA,
            pltpu.SemaphoreType.DMA,
            pltpu.SemaphoreType.DMA,
        ],
    )(_sc_body)
    sc_partials = sc(x_t, yt)

    tt = jnp.clip(yt, 0, _C - 1)
    v = jnp.take_along_axis(y_pred, tt[:, None], axis=1)[:, 0]
    tc_counts = pl.pallas_call(
        _tc_body,
        grid=(_GT,),
        in_specs=[
            pl.BlockSpec((1, _B), lambda i: (0, 0)),
            pl.BlockSpec((1, _B), lambda i: (0, 0)),
            pl.BlockSpec((_RT, _B), lambda i: (i + _TOFF, 0)),
        ],
        out_specs=pl.BlockSpec((1, _B), lambda i: (0, 0)),
        out_shape=jax.ShapeDtypeStruct((1, _B), jnp.int32),
    )(v[None, :], tt[None, :], x_t)

    counts = sc_partials.sum(axis=0) + tc_counts[0]
    valid = y_true != _IGNORE
    hits = ((counts < _K) & valid).astype(jnp.float32)
    w = valid.astype(jnp.float32)
    return (hits.sum() / w.sum()) * 100.0


# speedup vs baseline: 4.6462x; 1.1806x over previous
import functools

import jax
import jax.numpy as jnp
from jax import lax
from jax.experimental import pallas as pl
from jax.experimental.pallas import tpu as pltpu
from jax.experimental.pallas import tpu_sc as plsc

_K = 5
_IGNORE = -100
_B = 128
_C = 100000
_LANES = 16
_NC = 2
_NS = 16
_NW = _NC * _NS
_KV = _B // _LANES

_S_TC = 51360
_SLAB = (_C - _S_TC) // _NW
_RCH = 304
_NCH = _SLAB // _RCH

_RT = 2568
_GT = _S_TC // _RT


def _nextup(v):
    bi = lax.bitcast_convert_type(v, jnp.int32)
    bp = jnp.where(bi < 0, bi - 1, bi + 1)
    bp = jnp.where(bi == jnp.int32(-2147483648), jnp.int32(1), bp)
    return lax.bitcast_convert_type(bp, jnp.float32)


def _sc_body(x_hbm, ytrue_hbm, cnt_hbm,
             yt_v, idx_v, g_v, buf0, buf1, cnt_v,
             sem0, sem1, gsem):
    wid = lax.axis_index("s") * _NC + lax.axis_index("c")
    s0 = _S_TC + wid * _SLAB
    bufs = (buf0, buf1)
    sems = (sem0, sem1)

    def chunk_copy(c):
        off = pl.multiple_of(s0 + c * _RCH, 8)
        return pltpu.async_copy(x_hbm.at[pl.ds(off, _RCH)],
                                bufs[c % 2], sems[c % 2])

    cps = [chunk_copy(0), chunk_copy(1)]

    pltpu.sync_copy(ytrue_hbm, yt_v)
    lane_iota = lax.iota(jnp.int32, _LANES)
    for k in range(_KV):
        idx_v[pl.ds(k * _LANES, _LANES)] = jnp.clip(
            yt_v[pl.ds(k * _LANES, _LANES)], 0, _C - 1)
    pltpu.async_copy(x_hbm.at[idx_v], g_v, gsem).wait()

    ts, vps, thr0s = [], [], []
    for k in range(_KV):
        i16 = k * _LANES + lane_iota
        t_k = yt_v[pl.ds(k * _LANES, _LANES)]
        v_k = plsc.load_gather(g_v, [i16, i16])
        ts.append(t_k)
        vps.append(_nextup(v_k))
        thr0s.append(v_k)

    zero_i = jnp.zeros((_LANES,), jnp.int32)
    accs = (zero_i,) * _KV

    def fast_chunk(buf, c0, accs):
        thrs = [jnp.where(ts[k] >= c0 + _RCH, thr0s[k], vps[k])
                for k in range(_KV)]

        def step(rr, a):
            row = buf.at[rr]
            out = []
            for k in range(_KV):
                x = row[pl.ds(k * _LANES, _LANES)]
                out.append(a[k] + (x >= thrs[k]).astype(jnp.int32))
            return tuple(out)
        return lax.fori_loop(0, _RCH, step, accs)

    def slow_chunk(buf, c0, nrows, accs):
        thrs = tuple(jnp.where(ts[k] > c0, thr0s[k], vps[k])
                     for k in range(_KV))

        def step(rr, carry):
            a, th = carry
            j = jnp.full((_LANES,), c0, jnp.int32) + rr
            row = buf.at[rr]
            na, nth = [], []
            for k in range(_KV):
                x = row[pl.ds(k * _LANES, _LANES)]
                tk = jnp.where(j == ts[k], vps[k], th[k])
                na.append(a[k] + (x >= tk).astype(jnp.int32))
                nth.append(tk)
            return tuple(na), tuple(nth)
        accs, _ = lax.fori_loop(0, nrows, step, (accs, thrs))
        return accs

    for c in range(_NCH):
        cps[c].wait()
        c0 = s0 + c * _RCH
        c0s = jnp.full((_LANES,), c0, jnp.int32)
        inb = zero_i
        for k in range(_KV):
            inb = inb + plsc.all_reduce_population_count(
                (ts[k] >= c0s) & (ts[k] < c0s + _RCH))
        has_t = jnp.max(inb) > 0
        buf = bufs[c % 2]
        accs = lax.cond(
            has_t,
            functools.partial(slow_chunk, buf, c0, _RCH),
            functools.partial(fast_chunk, buf, c0),
            accs)
        if c + 2 < _NCH:
            cps.append(chunk_copy(c + 2))

    for k in range(_KV):
        cnt_v[pl.ds(k * _LANES, _LANES)] = accs[k]
    pltpu.sync_copy(cnt_v, cnt_hbm.at[wid])


def _tc_body(v_ref, t_ref, x_ref, o_ref):
    i = pl.program_id(0)

    @pl.when(i == 0)
    def _():
        o_ref[...] = jnp.zeros_like(o_ref)

    x = x_ref[...]
    v = v_ref[...]
    t = t_ref[...]
    j = lax.broadcasted_iota(jnp.int32, (_RT, _B), 0) + i * _RT
    m = (x > v) | ((x == v) & (j < t))
    o_ref[...] += m.astype(jnp.int32).sum(axis=0, keepdims=True)


@jax.jit
def kernel(y_pred, y_true):
    yt = y_true.astype(jnp.int32)
    x_t = y_pred.T

    mesh = plsc.VectorSubcoreMesh(core_axis_name="c", subcore_axis_name="s")
    sc = functools.partial(
        pl.kernel,
        mesh=mesh,
        compiler_params=pltpu.CompilerParams(needs_layout_passes=False),
        out_type=jax.ShapeDtypeStruct((_NW, _B), jnp.int32),
        scratch_types=[
            pltpu.VMEM((_B,), jnp.int32),
            pltpu.VMEM((_B,), jnp.int32),
            pltpu.VMEM((_B, _B), jnp.float32),
            pltpu.VMEM((_RCH, _B), jnp.float32),
            pltpu.VMEM((_RCH, _B), jnp.float32),
            pltpu.VMEM((_B,), jnp.int32),
            pltpu.SemaphoreType.DMA,
            pltpu.SemaphoreType.DMA,
            pltpu.SemaphoreType.DMA,
        ],
    )(_sc_body)
    sc_partials = sc(x_t, yt)

    tt = jnp.clip(yt, 0, _C - 1)
    v = jnp.take_along_axis(y_pred, tt[:, None], axis=1)[:, 0]
    tc_counts = pl.pallas_call(
        _tc_body,
        grid=(_GT,),
        in_specs=[
            pl.BlockSpec((1, _B), lambda i: (0, 0)),
            pl.BlockSpec((1, _B), lambda i: (0, 0)),
            pl.BlockSpec((_RT, _B), lambda i: (i, 0)),
        ],
        out_specs=pl.BlockSpec((1, _B), lambda i: (0, 0)),
        out_shape=jax.ShapeDtypeStruct((1, _B), jnp.int32),
    )(v[None, :], tt[None, :], x_t)

    counts = sc_partials.sum(axis=0) + tc_counts[0]
    valid = y_true != _IGNORE
    hits = ((counts < _K) & valid).astype(jnp.float32)
    w = valid.astype(jnp.float32)
    return (hits.sum() / w.sum()) * 100.0
